# baseline (speedup 1.0000x reference)
import functools

import jax
import jax.numpy as jnp
from jax import lax
from jax.experimental import pallas as pl
from jax.experimental.pallas import tpu as pltpu
from jax.experimental.pallas import tpu_sc as plsc

_B, _T, _EMB = 4096, 200, 64
_NTOT = _B * _T
_NW = 32
_RPW = _NTOT // _NW
_G = 128
_K = 4
_C = _G * _K
_NCH = _RPW // _C

_mesh = plsc.VectorSubcoreMesh(core_axis_name="c", subcore_axis_name="s")


@functools.partial(
    pl.kernel,
    out_type=jax.ShapeDtypeStruct((_NTOT, _EMB), jnp.float32),
    mesh=_mesh,
    scratch_types=[
        pltpu.VMEM((_K, _G), jnp.int32),
        pltpu.VMEM((_C, _EMB), jnp.float32),
        pltpu.SemaphoreType.DMA,
    ],
    compiler_params=pltpu.CompilerParams(use_tc_tiling_on_sc=False),
)
def _sc_gather(x_hbm, table_hbm, out_hbm, idx_v, rows_v, sem):
    nc = plsc.get_sparse_core_info().num_cores
    wid = lax.axis_index("s") * nc + lax.axis_index("c")
    row0 = wid * (_RPW // _G)

    def chunk(g, carry):
        blk = row0 + g * _K
        pltpu.sync_copy(x_hbm.at[pl.ds(blk, _K)], idx_v)
        copies = [
            pltpu.async_copy(
                table_hbm.at[idx_v.at[j]],
                rows_v.at[pl.ds(j * _G, _G)],
                sem,
            )
            for j in range(_K)
        ]
        for c in copies:
            c.wait()
        pltpu.sync_copy(rows_v, out_hbm.at[pl.ds(blk * _G, _C)])
        return carry

    lax.fori_loop(0, _NCH, chunk, 0)


def kernel(x, table):
    x2d = x.reshape(_NTOT // _G, _G)
    out = _sc_gather(x2d, table)
    return out.reshape(_B, _T, _EMB)


# speedup vs baseline: 1.0414x; 1.0414x over previous
import functools

import jax
import jax.numpy as jnp
from jax import lax
from jax.experimental import pallas as pl
from jax.experimental.pallas import tpu as pltpu
from jax.experimental.pallas import tpu_sc as plsc

_B, _T, _EMB = 4096, 200, 64
_NTOT = _B * _T
_NW = 32
_RPW = _NTOT // _NW
_G = 128
_K = 4
_C = _G * _K
_NCH = _RPW // _C
_NBLK = _RPW // _G

_mesh = plsc.VectorSubcoreMesh(core_axis_name="c", subcore_axis_name="s")


@functools.partial(
    pl.kernel,
    out_type=jax.ShapeDtypeStruct((_NTOT, _EMB), jnp.float32),
    mesh=_mesh,
    scratch_types=[
        pltpu.VMEM((_NBLK, _G), jnp.int32),
        pltpu.VMEM((_C, _EMB), jnp.float32),
        pltpu.VMEM((_C, _EMB), jnp.float32),
        pltpu.SemaphoreType.DMA,
        pltpu.SemaphoreType.DMA,
        pltpu.SemaphoreType.DMA,
        pltpu.SemaphoreType.DMA,
    ],
    compiler_params=pltpu.CompilerParams(use_tc_tiling_on_sc=False),
)
def _sc_gather(x_hbm, table_hbm, out_hbm, idx_all, rows0, rows1, g0, g1, s0, s1):
    nc = plsc.get_sparse_core_info().num_cores
    wid = lax.axis_index("s") * nc + lax.axis_index("c")
    row0 = wid * _NBLK
    base = row0 * _G

    rows = (rows0, rows1)
    gsem = (g0, g1)
    ssem = (s0, s1)

    pltpu.sync_copy(x_hbm.at[pl.ds(row0, _NBLK)], idx_all)

    def fire_gather(g, b):
        for j in range(_K):
            pltpu.async_copy(
                table_hbm.at[idx_all.at[g * _K + j]],
                rows[b].at[pl.ds(j * _G, _G)],
                gsem[b],
            )

    def wait_gather(b):
        pltpu.make_async_copy(out_hbm.at[pl.ds(0, _C)], rows[b], gsem[b]).wait()

    def fire_store(g, b):
        pltpu.async_copy(rows[b], out_hbm.at[pl.ds(base + g * _C, _C)], ssem[b])

    def wait_store(b):
        pltpu.make_async_copy(rows[b], out_hbm.at[pl.ds(0, _C)], ssem[b]).wait()

    fire_gather(0, 0)
    fire_gather(1, 1)

    def body(i, carry):
        for b in range(2):
            g = 2 * i + b
            wait_gather(b)
            fire_store(g, b)
            wait_store(b)
            fire_gather(g + 2, b)
        return carry

    lax.fori_loop(0, (_NCH - 2) // 2, body, 0)

    for b in range(2):
        g = _NCH - 2 + b
        wait_gather(b)
        fire_store(g, b)
    for b in range(2):
        wait_store(b)


def kernel(x, table):
    x2d = x.reshape(_NTOT // _G, _G)
    out = _sc_gather(x2d, table)
    return out.reshape(_B, _T, _EMB)
